# baseline (speedup 1.0000x reference)
import functools

import jax
import jax.numpy as jnp
from jax import lax
from jax.experimental import pallas as pl
from jax.experimental.pallas import tpu as pltpu
from jax.experimental.pallas import tpu_sc as plsc

L = 16

OFF_XLR = 0
OFF_XLT = 768
OFF_OHT = 1536
OFF_WL = 2064
OFF_BL = 3088
OFF_WF = 3152
OFF_BF = 4176
OFF_WGR = 4192
OFF_BGR = 4896
OFF_A = 4960
OFF_B = 5024
PF = 5040

OFF_ESRC = 0
OFF_EDST = 112
OFF_BP = 224
PI = 272

N_EDGES = 47
N_STEPS = 2 * N_EDGES - 1
NL = 48
STOP = 48
NCLS = 11


def _vlog(x):
    bits = plsc.bitcast(x, jnp.int32)
    e = ((bits >> 23) & 255) - 127
    m = plsc.bitcast((bits & 0x007FFFFF) | 0x3F800000, jnp.float32)
    big = m > 1.4142135623730951
    m = jnp.where(big, m * 0.5, m)
    ef = e.astype(jnp.float32) + jnp.where(big, 1.0, 0.0)
    t = m - 1.0
    s = t / (t + 2.0)
    z2 = s * s
    p = s * (2.0 + z2 * (0.6666666666666666 + z2 * (0.4 + z2 * (
        0.2857142857142857 + z2 * 0.2222222222222222))))
    return ef * 0.6931471805599453 + p


def _make_sc_kernel():
    mesh = plsc.VectorSubcoreMesh(core_axis_name="c", subcore_axis_name="s",
                                  num_cores=2, num_subcores=16)

    @functools.partial(
        pl.kernel,
        out_type=jax.ShapeDtypeStruct((L,), jnp.float32),
        mesh=mesh,
        compiler_params=pltpu.CompilerParams(needs_layout_passes=False),
        scratch_types=[
            pltpu.VMEM((PF,), jnp.float32),
            pltpu.VMEM((PI,), jnp.int32),
            pltpu.VMEM((NL,), jnp.int32),
            pltpu.VMEM((768,), jnp.float32),
            pltpu.VMEM((768,), jnp.float32),
            pltpu.VMEM((704,), jnp.float32),
            pltpu.VMEM((704,), jnp.float32),
            pltpu.VMEM((704,), jnp.float32),
            pltpu.VMEM((64,), jnp.float32),
            pltpu.VMEM((64,), jnp.float32),
            pltpu.VMEM((64,), jnp.float32),
            pltpu.VMEM((L,), jnp.float32),
        ],
    )
    def sc_kernel(pf_hbm, pi_hbm, out_hbm, pf, pi, srcsv, aggR, hT, xlabT,
                  sumLT, hLT, degL, clogv, blogv, outv):
        cid = lax.axis_index("c")
        sid = lax.axis_index("s")

        @pl.when(jnp.logical_and(cid == 0, sid == 0))
        def _body():
            iota = lax.iota(jnp.int32, L)
            zf = jnp.zeros((L,), jnp.float32)
            onef = jnp.ones((L,), jnp.float32)
            lane0 = iota == 0
            lane01 = iota < 2

            def bci(x):
                return jnp.full((L,), x, jnp.int32)

            def bcf(x):
                return jnp.full((L,), x, jnp.float32)

            def lanebc(vec, lane):
                return jnp.take_along_axis(vec, bci(lane), axis=0)

            lt11 = iota < 11
            lt6 = iota < 6
            pat1 = 64 * jnp.where(lt11, iota, iota - 11)
            pat2 = 64 * jnp.where(lt6, iota + 5, 10)
            cpack = 64 * jnp.minimum(iota, 10)

            pltpu.sync_copy(pf_hbm, pf)
            pltpu.sync_copy(pi_hbm, pi)

            for i in range(48):
                aggR[pl.ds(L * i, L)] = zf
            for i in range(44):
                sumLT[pl.ds(L * i, L)] = zf
                xlabT[pl.ds(L * i, L)] = zf
            for i in range(4):
                degL[pl.ds(L * i, L)] = onef

            for i in range(3):
                bp = pi[pl.ds(OFF_BP + L * i, L)]
                srcsv[pl.ds(L * i, L)] = bp % (iota + (L * i + 1))

            def edge_body(e, carry):
                sv = plsc.load_gather(pi, [bci(OFF_ESRC) + bci(e)])
                dv = plsc.load_gather(pi, [bci(OFF_EDST) + bci(e)])
                vals = plsc.load_gather(pf, [sv * L + (OFF_XLR + iota)])
                plsc.addupdate_scatter(aggR, [dv * L + iota], vals)
                return carry

            lax.fori_loop(0, 100, edge_body, jnp.int32(0))

            rdeg = []
            for r in range(3):
                d = plsc.load_gather(aggR, [(iota + L * r) * L + 15]) + 1.0
                rdeg.append(1.0 / d)
            for k in range(15):
                for r in range(3):
                    aggcol = plsc.load_gather(aggR, [(iota + L * r) * L + k])
                    xlcol = pf[pl.ds(OFF_XLT + 48 * k + L * r, L)]
                    hT[pl.ds(48 * k + L * r, L)] = (aggcol + xlcol) * rdeg[r]

            def zbody(jj, carry):
                j0 = bci(2 * jj)
                j1 = j0 + 1
                wl0 = plsc.load_gather(pf, [(OFF_WL + 64 * iota) + j0])
                wl1 = plsc.load_gather(pf, [(OFF_WL + 64 * iota) + j1])
                wf0 = plsc.load_gather(pf, [(OFF_WF + iota) + j0 * L])
                wf1 = plsc.load_gather(pf, [(OFF_WF + iota) + j1 * L])
                bl0 = plsc.load_gather(pf, [bci(OFF_BL) + j0])
                bl1 = plsc.load_gather(pf, [bci(OFF_BL) + j1])
                a0 = [bl0, bl0, bl0]
                a1 = [bl1, bl1, bl1]
                for k in range(15):
                    w0 = lanebc(wl0, k)
                    w1 = lanebc(wl1, k)
                    for r in range(3):
                        h = hT[pl.ds(48 * k + L * r, L)]
                        a0[r] = a0[r] + h * w0
                        a1[r] = a1[r] + h * w1
                a0 = [jnp.maximum(a, 0.0) for a in a0]
                a1 = [jnp.maximum(a, 0.0) for a in a1]
                out = []
                for c in range(NCLS):
                    w0 = lanebc(wf0, c)
                    w1 = lanebc(wf1, c)
                    for r in range(3):
                        out.append(carry[3 * c + r] + a0[r] * w0 + a1[r] * w1)
                return tuple(out)

            lacc = lax.fori_loop(0, 32, zbody, (zf,) * (3 * NCLS))

            lT = [[lacc[3 * c + r] + plsc.load_gather(pf, [bci(OFF_BF + c)])
                   for r in range(3)] for c in range(NCLS)]
            lp0 = jnp.float32(0.0)
            for r in range(3):
                mx = lT[0][r]
                for c in range(1, NCLS):
                    mx = jnp.maximum(mx, lT[c][r])
                ex = [jnp.exp(lT[c][r] - mx) for c in range(NCLS)]
                sden = ex[0]
                for c in range(1, NCLS):
                    sden = sden + ex[c]
                inner = zf
                for c in range(NCLS):
                    inner = inner + ex[c] * pf[pl.ds(OFF_OHT + 48 * c + L * r, L)]
                lp0 = lp0 + jnp.sum(_vlog(inner) - _vlog(sden))
                rs = 1.0 / sden
                for c in range(NCLS):
                    p = ex[c] * rs
                    xlabT[pl.ds(64 * c + L * r, L)] = p
                    sumLT[pl.ds(64 * c + L * r, L)] = p
            plsc.store_scatter(xlabT, [bci(64 * 10 + STOP)], onef, mask=lane0)
            plsc.store_scatter(sumLT, [bci(64 * 10 + STOP)], onef, mask=lane0)

            for q in range(4):
                acc = zf
                for c in range(NCLS):
                    acc = acc + xlabT[pl.ds(64 * c + L * q, L)] * \
                        plsc.load_gather(pf, [bci(OFF_B + c)])
                blogv[pl.ds(L * q, L)] = acc

            def full_recompute():
                rdq = [1.0 / degL[pl.ds(L * q, L)] for q in range(4)]
                for c in range(NCLS):
                    for q in range(4):
                        off = 64 * c + L * q
                        hLT[pl.ds(off, L)] = sumLT[pl.ds(off, L)] * rdq[q]

                offbga = jnp.where(lane0, OFF_BGR, OFF_A)

                def jbody(jj, carry):
                    j0 = bci(2 * jj)
                    j1 = j0 + 1
                    wg0 = plsc.load_gather(pf, [(OFF_WGR + cpack) + j0])
                    wg1 = plsc.load_gather(pf, [(OFF_WGR + cpack) + j1])
                    ba0 = plsc.load_gather(pf, [offbga + j0])
                    ba1 = plsc.load_gather(pf, [offbga + j1])
                    bg0 = lanebc(ba0, 0)
                    bg1 = lanebc(ba1, 0)
                    a0 = [bg0, bg0, bg0, bg0]
                    a1 = [bg1, bg1, bg1, bg1]
                    for c in range(NCLS):
                        w0 = lanebc(wg0, c)
                        w1 = lanebc(wg1, c)
                        for q in range(4):
                            h = hLT[pl.ds(64 * c + L * q, L)]
                            a0[q] = a0[q] + h * w0
                            a1[q] = a1[q] + h * w1
                    av0 = lanebc(ba0, 1)
                    av1 = lanebc(ba1, 1)
                    return tuple(carry[q] + jnp.maximum(a0[q], 0.0) * av0 +
                                 jnp.maximum(a1[q], 0.0) * av1
                                 for q in range(4))

                al = lax.fori_loop(0, 32, jbody, (zf,) * 4)
                for q in range(3):
                    clogv[pl.ds(L * q, L)] = al[q] + blogv[pl.ds(L * q, L)]
                clogv[pl.ds(48, L)] = jnp.where(
                    lane0, al[3] + blogv[pl.ds(48, L)], -1e30)

            s0b = plsc.load_gather(srcsv, [bci(0)])
            for c in range(NCLS):
                val = plsc.load_gather(xlabT, [bci(64 * c) + s0b])
                plsc.addupdate_scatter(sumLT, [bci(64 * c + 1)], val, mask=lane0)
            plsc.addupdate_scatter(degL, [bci(1)], onef, mask=lane0)
            full_recompute()
            plsc.store_scatter(clogv, [bci(STOP)],
                               plsc.load_gather(blogv, [bci(STOP)]), mask=lane0)
            for c in range(NCLS):
                val = plsc.load_gather(xlabT, [bci(64 * c) + s0b])
                plsc.addupdate_scatter(sumLT, [bci(64 * c + 1)], -val, mask=lane0)
            plsc.addupdate_scatter(degL, [bci(1)], -onef, mask=lane0)

            def step(idx, last, lp, first):
                u0 = jnp.max(plsc.load_gather(srcsv, [bci(idx)]))
                stopping = last != u0
                u = jnp.where(stopping, last, u0)
                v = jnp.where(stopping, jnp.int32(STOP), idx + 1)
                lv = [clogv[pl.ds(L * q, L)] for q in range(4)]
                mv = jnp.maximum(jnp.maximum(lv[0], lv[1]),
                                 jnp.maximum(lv[2], lv[3]))
                mb = bcf(jnp.max(mv))
                ev = (jnp.exp(lv[0] - mb) + jnp.exp(lv[1] - mb) +
                      jnp.exp(lv[2] - mb) + jnp.exp(lv[3] - mb))
                vb = bci(v)
                lpv = plsc.load_gather(clogv, [vb])
                lp = lp + lpv - mb - _vlog(bcf(jnp.sum(ev)))
                ub = bci(u)
                guv = jnp.where(lane0, ub, vb)
                g1 = plsc.load_gather(xlabT, [jnp.where(lt11, ub, vb) + pat1])
                plsc.addupdate_scatter(sumLT, [jnp.where(lt11, vb, ub) + pat1],
                                       g1)
                g2 = plsc.load_gather(xlabT, [vb + pat2])
                plsc.addupdate_scatter(sumLT, [ub + pat2], g2, mask=lt6)
                plsc.addupdate_scatter(degL, [guv], onef, mask=lane01)
                if first:
                    full_recompute()
                else:
                    rdp = 1.0 / plsc.load_gather(degL, [guv])
                    hpu = plsc.load_gather(sumLT, [cpack + ub]) * lanebc(rdp, 0)
                    hpv = plsc.load_gather(sumLT, [cpack + vb]) * lanebc(rdp, 1)
                    hu = [lanebc(hpu, c) for c in range(NCLS)]
                    hv = [lanebc(hpv, c) for c in range(NCLS)]
                    pu = zf
                    pv = zf
                    for q in range(4):
                        bg = pf[pl.ds(OFF_BGR + L * q, L)]
                        au = bg
                        av = bg
                        for c in range(NCLS):
                            w = pf[pl.ds(OFF_WGR + 64 * c + L * q, L)]
                            au = au + hu[c] * w
                            av = av + hv[c] * w
                        aq = pf[pl.ds(OFF_A + L * q, L)]
                        pu = pu + jnp.maximum(au, 0.0) * aq
                        pv = pv + jnp.maximum(av, 0.0) * aq
                    newlog = jnp.where(lane0, bcf(jnp.sum(pu)), bcf(jnp.sum(pv)))
                    newlog = newlog + plsc.load_gather(blogv, [guv])
                    plsc.store_scatter(clogv, [guv], newlog, mask=lane01)
                idx2 = jnp.where(stopping, idx, idx + 1)
                last2 = jnp.where(stopping, u0, last)
                return idx2, last2, lp

            idx0 = jnp.int32(0)
            last0 = jnp.max(s0b)
            idx1, last1, lp1 = step(idx0, last0, bcf(lp0), True)

            def wcond(carry):
                idx, _, _, it = carry
                return jnp.logical_and(idx < N_EDGES, it < N_STEPS)

            def wbody(carry):
                idx, last, lp, it = carry
                i2, l2, lp2 = step(idx, last, lp, False)
                return (i2, l2, lp2, it + 1)

            _, _, lpf, _ = lax.while_loop(
                wcond, wbody, (idx1, last1, lp1, jnp.int32(1)))

            outv[...] = lpf
            pltpu.sync_copy(outv, out_hbm)

    return sc_kernel


_SC_KERNEL_CACHE = []


def _get_sc_kernel():
    if not _SC_KERNEL_CACHE:
        _SC_KERNEL_CACHE.append(_make_sc_kernel())
    return _SC_KERNEL_CACHE[0]


def kernel(x_p, edge_index_p, x_l, edge_index_l, bfs_parent, Wp, bp, Wl, bl,
           Wgr, bgr, Wf, bf, Wg, bg):
    f32 = jnp.float32
    xlr = jnp.concatenate([x_l.astype(f32),
                           jnp.ones((NL, 1), f32)], axis=1)
    xlt = xlr.T
    oht = x_l[:, 4:].astype(f32).T
    wl = jnp.concatenate([Wl.astype(f32), jnp.zeros((1, 64), f32)], axis=0)
    wf = jnp.concatenate([Wf.astype(f32), jnp.zeros((64, 5), f32)], axis=1)
    bf16 = jnp.concatenate([bf.astype(f32), jnp.zeros((5,), f32)])
    a_vec = Wg[204:268, 0].astype(f32)
    b_vec = jnp.concatenate([Wg[268:279, 0].astype(f32), jnp.zeros((5,), f32)])
    pf = jnp.concatenate([
        xlr.reshape(-1), xlt.reshape(-1), oht.reshape(-1), wl.reshape(-1),
        bl.astype(f32), wf.reshape(-1), bf16, Wgr.astype(f32).reshape(-1),
        bgr.astype(f32), a_vec, b_vec,
    ])
    i32 = jnp.int32
    esrc = jnp.concatenate([edge_index_l[0].astype(i32),
                            jnp.zeros((12,), i32)])
    edst = jnp.concatenate([edge_index_l[1].astype(i32),
                            jnp.zeros((12,), i32)])
    bpv = jnp.concatenate([bfs_parent.astype(i32), jnp.zeros((1,), i32)])
    pi = jnp.concatenate([esrc, edst, bpv])
    out = _get_sc_kernel()(pf, pi)
    return out[0]


# speedup vs baseline: 1.0218x; 1.0218x over previous
import functools

import jax
import jax.numpy as jnp
from jax import lax
from jax.experimental import pallas as pl
from jax.experimental.pallas import tpu as pltpu
from jax.experimental.pallas import tpu_sc as plsc

L = 16

OFF_XLR = 0
OFF_XLT = 768
OFF_OHT = 1536
OFF_WL = 2064
OFF_BL = 3088
OFF_WF = 3152
OFF_BF = 4176
OFF_WGR = 4192
OFF_BGR = 4896
OFF_A = 4960
OFF_B = 5024
PF = 5040

OFF_ESRC = 0
OFF_EDST = 112
OFF_BP = 224
PI = 272

N_EDGES = 47
N_STEPS = 2 * N_EDGES - 1
NL = 48
STOP = 48
NCLS = 11


def _vlog(x):
    bits = plsc.bitcast(x, jnp.int32)
    e = ((bits >> 23) & 255) - 127
    m = plsc.bitcast((bits & 0x007FFFFF) | 0x3F800000, jnp.float32)
    big = m > 1.4142135623730951
    m = jnp.where(big, m * 0.5, m)
    ef = e.astype(jnp.float32) + jnp.where(big, 1.0, 0.0)
    t = m - 1.0
    s = t / (t + 2.0)
    z2 = s * s
    p = s * (2.0 + z2 * (0.6666666666666666 + z2 * (0.4 + z2 * (
        0.2857142857142857 + z2 * 0.2222222222222222))))
    return ef * 0.6931471805599453 + p


def _make_sc_kernel():
    mesh = plsc.VectorSubcoreMesh(core_axis_name="c", subcore_axis_name="s",
                                  num_cores=2, num_subcores=16)

    @functools.partial(
        pl.kernel,
        out_type=jax.ShapeDtypeStruct((L,), jnp.float32),
        mesh=mesh,
        compiler_params=pltpu.CompilerParams(needs_layout_passes=False),
        scratch_types=[
            pltpu.VMEM((PF,), jnp.float32),
            pltpu.VMEM((PI,), jnp.int32),
            pltpu.VMEM((NL,), jnp.int32),
            pltpu.VMEM((768,), jnp.float32),
            pltpu.VMEM((768,), jnp.float32),
            pltpu.VMEM((704,), jnp.float32),
            pltpu.VMEM((704,), jnp.float32),
            pltpu.VMEM((704,), jnp.float32),
            pltpu.VMEM((64,), jnp.float32),
            pltpu.VMEM((64,), jnp.float32),
            pltpu.VMEM((64,), jnp.float32),
            pltpu.VMEM((L,), jnp.float32),
        ],
    )
    def sc_kernel(pf_hbm, pi_hbm, out_hbm, pf, pi, srcsv, aggR, hT, xlabT,
                  sumLT, hLT, degL, clogv, blogv, outv):
        cid = lax.axis_index("c")
        sid = lax.axis_index("s")

        @pl.when(jnp.logical_and(cid == 0, sid == 0))
        def _body():
            iota = lax.iota(jnp.int32, L)
            zf = jnp.zeros((L,), jnp.float32)
            onef = jnp.ones((L,), jnp.float32)
            lane0 = iota == 0
            lane01 = iota < 2

            def bci(x):
                return jnp.full((L,), x, jnp.int32)

            def bcf(x):
                return jnp.full((L,), x, jnp.float32)

            def lanebc(vec, lane):
                return jnp.take_along_axis(vec, bci(lane), axis=0)

            lt11 = iota < 11
            lt6 = iota < 6
            pat1 = 64 * jnp.where(lt11, iota, iota - 11)
            pat2 = 64 * jnp.where(lt6, iota + 5, 10)
            cpack = 64 * jnp.minimum(iota, 10)

            pltpu.sync_copy(pf_hbm, pf)
            pltpu.sync_copy(pi_hbm, pi)

            for i in range(48):
                aggR[pl.ds(L * i, L)] = zf
            for i in range(44):
                sumLT[pl.ds(L * i, L)] = zf
                xlabT[pl.ds(L * i, L)] = zf
            for i in range(4):
                degL[pl.ds(L * i, L)] = onef

            for i in range(3):
                bp = pi[pl.ds(OFF_BP + L * i, L)]
                srcsv[pl.ds(L * i, L)] = bp % (iota + (L * i + 1))

            def edge_body(e, carry):
                sv = plsc.load_gather(pi, [bci(OFF_ESRC) + bci(e)])
                dv = plsc.load_gather(pi, [bci(OFF_EDST) + bci(e)])
                vals = plsc.load_gather(pf, [sv * L + (OFF_XLR + iota)])
                plsc.addupdate_scatter(aggR, [dv * L + iota], vals)
                return carry

            lax.fori_loop(0, 100, edge_body, jnp.int32(0))

            rdeg = []
            for r in range(3):
                d = plsc.load_gather(aggR, [(iota + L * r) * L + 15]) + 1.0
                rdeg.append(1.0 / d)
            for k in range(15):
                for r in range(3):
                    aggcol = plsc.load_gather(aggR, [(iota + L * r) * L + k])
                    xlcol = pf[pl.ds(OFF_XLT + 48 * k + L * r, L)]
                    hT[pl.ds(48 * k + L * r, L)] = (aggcol + xlcol) * rdeg[r]

            def zbody(j, carry):
                jb = bci(j)
                wl0 = plsc.load_gather(pf, [(OFF_WL + 64 * iota) + jb])
                wf0 = plsc.load_gather(pf, [(OFF_WF + iota) + jb * L])
                bl0 = plsc.load_gather(pf, [bci(OFF_BL) + jb])
                a0 = [bl0, bl0, bl0]
                for k in range(15):
                    w0 = lanebc(wl0, k)
                    for r in range(3):
                        a0[r] = a0[r] + hT[pl.ds(48 * k + L * r, L)] * w0
                a0 = [jnp.maximum(a, 0.0) for a in a0]
                out = []
                for c in range(NCLS):
                    w0 = lanebc(wf0, c)
                    for r in range(3):
                        out.append(carry[3 * c + r] + a0[r] * w0)
                return tuple(out)

            lacc = lax.fori_loop(0, 64, zbody, (zf,) * (3 * NCLS))

            lT = [[lacc[3 * c + r] + plsc.load_gather(pf, [bci(OFF_BF + c)])
                   for r in range(3)] for c in range(NCLS)]
            lp0 = jnp.float32(0.0)
            for r in range(3):
                mx = lT[0][r]
                for c in range(1, NCLS):
                    mx = jnp.maximum(mx, lT[c][r])
                ex = [jnp.exp(lT[c][r] - mx) for c in range(NCLS)]
                sden = ex[0]
                for c in range(1, NCLS):
                    sden = sden + ex[c]
                inner = zf
                for c in range(NCLS):
                    inner = inner + ex[c] * pf[pl.ds(OFF_OHT + 48 * c + L * r, L)]
                lp0 = lp0 + jnp.sum(_vlog(inner) - _vlog(sden))
                rs = 1.0 / sden
                for c in range(NCLS):
                    p = ex[c] * rs
                    xlabT[pl.ds(64 * c + L * r, L)] = p
                    sumLT[pl.ds(64 * c + L * r, L)] = p
            plsc.store_scatter(xlabT, [bci(64 * 10 + STOP)], onef, mask=lane0)
            plsc.store_scatter(sumLT, [bci(64 * 10 + STOP)], onef, mask=lane0)

            for q in range(4):
                acc = zf
                for c in range(NCLS):
                    acc = acc + xlabT[pl.ds(64 * c + L * q, L)] * \
                        plsc.load_gather(pf, [bci(OFF_B + c)])
                blogv[pl.ds(L * q, L)] = acc

            def full_recompute():
                rdq = [1.0 / degL[pl.ds(L * q, L)] for q in range(4)]
                for c in range(NCLS):
                    for q in range(4):
                        off = 64 * c + L * q
                        hLT[pl.ds(off, L)] = sumLT[pl.ds(off, L)] * rdq[q]

                offbga = jnp.where(lane0, OFF_BGR, OFF_A)

                def jbody(j, carry):
                    j0 = bci(j)
                    wg0 = plsc.load_gather(pf, [(OFF_WGR + cpack) + j0])
                    ba0 = plsc.load_gather(pf, [offbga + j0])
                    bg0 = lanebc(ba0, 0)
                    a0 = [bg0, bg0, bg0, bg0]
                    for c in range(NCLS):
                        w0 = lanebc(wg0, c)
                        for q in range(4):
                            a0[q] = a0[q] + hLT[pl.ds(64 * c + L * q, L)] * w0
                    av0 = lanebc(ba0, 1)
                    return tuple(carry[q] + jnp.maximum(a0[q], 0.0) * av0
                                 for q in range(4))

                al = lax.fori_loop(0, 64, jbody, (zf,) * 4)
                for q in range(3):
                    clogv[pl.ds(L * q, L)] = al[q] + blogv[pl.ds(L * q, L)]
                clogv[pl.ds(48, L)] = jnp.where(
                    lane0, al[3] + blogv[pl.ds(48, L)], -1e30)

            s0b = plsc.load_gather(srcsv, [bci(0)])
            for c in range(NCLS):
                val = plsc.load_gather(xlabT, [bci(64 * c) + s0b])
                plsc.addupdate_scatter(sumLT, [bci(64 * c + 1)], val, mask=lane0)
            plsc.addupdate_scatter(degL, [bci(1)], onef, mask=lane0)
            full_recompute()
            plsc.store_scatter(clogv, [bci(STOP)],
                               plsc.load_gather(blogv, [bci(STOP)]), mask=lane0)
            for c in range(NCLS):
                val = plsc.load_gather(xlabT, [bci(64 * c) + s0b])
                plsc.addupdate_scatter(sumLT, [bci(64 * c + 1)], -val, mask=lane0)
            plsc.addupdate_scatter(degL, [bci(1)], -onef, mask=lane0)

            def step(idx, last, lp, first):
                u0 = jnp.max(plsc.load_gather(srcsv, [bci(idx)]))
                stopping = last != u0
                u = jnp.where(stopping, last, u0)
                v = jnp.where(stopping, jnp.int32(STOP), idx + 1)
                lv = [clogv[pl.ds(L * q, L)] for q in range(4)]
                mv = jnp.maximum(jnp.maximum(lv[0], lv[1]),
                                 jnp.maximum(lv[2], lv[3]))
                mb = bcf(jnp.max(mv))
                ev = (jnp.exp(lv[0] - mb) + jnp.exp(lv[1] - mb) +
                      jnp.exp(lv[2] - mb) + jnp.exp(lv[3] - mb))
                vb = bci(v)
                lpv = plsc.load_gather(clogv, [vb])
                lp = lp + lpv - mb - _vlog(bcf(jnp.sum(ev)))
                ub = bci(u)
                guv = jnp.where(lane0, ub, vb)
                g1 = plsc.load_gather(xlabT, [jnp.where(lt11, ub, vb) + pat1])
                plsc.addupdate_scatter(sumLT, [jnp.where(lt11, vb, ub) + pat1],
                                       g1)
                g2 = plsc.load_gather(xlabT, [vb + pat2])
                plsc.addupdate_scatter(sumLT, [ub + pat2], g2, mask=lt6)
                plsc.addupdate_scatter(degL, [guv], onef, mask=lane01)
                if first:
                    full_recompute()
                else:
                    rdp = 1.0 / plsc.load_gather(degL, [guv])
                    hpu = plsc.load_gather(sumLT, [cpack + ub]) * lanebc(rdp, 0)
                    hpv = plsc.load_gather(sumLT, [cpack + vb]) * lanebc(rdp, 1)
                    hu = [lanebc(hpu, c) for c in range(NCLS)]
                    hv = [lanebc(hpv, c) for c in range(NCLS)]
                    pu = zf
                    pv = zf
                    for q in range(4):
                        bg = pf[pl.ds(OFF_BGR + L * q, L)]
                        au = bg
                        av = bg
                        for c in range(NCLS):
                            w = pf[pl.ds(OFF_WGR + 64 * c + L * q, L)]
                            au = au + hu[c] * w
                            av = av + hv[c] * w
                        aq = pf[pl.ds(OFF_A + L * q, L)]
                        pu = pu + jnp.maximum(au, 0.0) * aq
                        pv = pv + jnp.maximum(av, 0.0) * aq
                    newlog = jnp.where(lane0, bcf(jnp.sum(pu)), bcf(jnp.sum(pv)))
                    newlog = newlog + plsc.load_gather(blogv, [guv])
                    plsc.store_scatter(clogv, [guv], newlog, mask=lane01)
                idx2 = jnp.where(stopping, idx, idx + 1)
                last2 = jnp.where(stopping, u0, last)
                return idx2, last2, lp

            idx0 = jnp.int32(0)
            last0 = jnp.max(s0b)
            idx1, last1, lp1 = step(idx0, last0, bcf(lp0), True)

            def wcond(carry):
                idx, _, _, it = carry
                return jnp.logical_and(idx < N_EDGES, it < N_STEPS)

            def wbody(carry):
                idx, last, lp, it = carry
                i2, l2, lp2 = step(idx, last, lp, False)
                return (i2, l2, lp2, it + 1)

            _, _, lpf, _ = lax.while_loop(
                wcond, wbody, (idx1, last1, lp1, jnp.int32(1)))

            outv[...] = lpf
            pltpu.sync_copy(outv, out_hbm)

    return sc_kernel


_SC_KERNEL_CACHE = []


def _get_sc_kernel():
    if not _SC_KERNEL_CACHE:
        _SC_KERNEL_CACHE.append(_make_sc_kernel())
    return _SC_KERNEL_CACHE[0]


def kernel(x_p, edge_index_p, x_l, edge_index_l, bfs_parent, Wp, bp, Wl, bl,
           Wgr, bgr, Wf, bf, Wg, bg):
    f32 = jnp.float32
    xlr = jnp.concatenate([x_l.astype(f32),
                           jnp.ones((NL, 1), f32)], axis=1)
    xlt = xlr.T
    oht = x_l[:, 4:].astype(f32).T
    wl = jnp.concatenate([Wl.astype(f32), jnp.zeros((1, 64), f32)], axis=0)
    wf = jnp.concatenate([Wf.astype(f32), jnp.zeros((64, 5), f32)], axis=1)
    bf16 = jnp.concatenate([bf.astype(f32), jnp.zeros((5,), f32)])
    a_vec = Wg[204:268, 0].astype(f32)
    b_vec = jnp.concatenate([Wg[268:279, 0].astype(f32), jnp.zeros((5,), f32)])
    pf = jnp.concatenate([
        xlr.reshape(-1), xlt.reshape(-1), oht.reshape(-1), wl.reshape(-1),
        bl.astype(f32), wf.reshape(-1), bf16, Wgr.astype(f32).reshape(-1),
        bgr.astype(f32), a_vec, b_vec,
    ])
    i32 = jnp.int32
    esrc = jnp.concatenate([edge_index_l[0].astype(i32),
                            jnp.zeros((12,), i32)])
    edst = jnp.concatenate([edge_index_l[1].astype(i32),
                            jnp.zeros((12,), i32)])
    bpv = jnp.concatenate([bfs_parent.astype(i32), jnp.zeros((1,), i32)])
    pi = jnp.concatenate([esrc, edst, bpv])
    out = _get_sc_kernel()(pf, pi)
    return out[0]


# speedup vs baseline: 1.1441x; 1.1197x over previous
import functools

import jax
import jax.numpy as jnp
from jax import lax
from jax.experimental import pallas as pl
from jax.experimental.pallas import tpu as pltpu
from jax.experimental.pallas import tpu_sc as plsc

L = 16

OFF_XLR = 0
OFF_XLT = 768
OFF_OHT = 1536
OFF_WL = 2064
OFF_BL = 3088
OFF_WF = 3152
OFF_BF = 4176
OFF_WGR = 4192
OFF_BGR = 4896
OFF_A = 4960
OFF_B = 5024
PF = 5040

OFF_ESRC = 0
OFF_EDST = 112
OFF_BP = 224
PI = 272

N_EDGES = 47
N_STEPS = 2 * N_EDGES - 1
NL = 48
STOP = 48
NCLS = 11


def _vlog(x):
    bits = plsc.bitcast(x, jnp.int32)
    e = ((bits >> 23) & 255) - 127
    m = plsc.bitcast((bits & 0x007FFFFF) | 0x3F800000, jnp.float32)
    big = m > 1.4142135623730951
    m = jnp.where(big, m * 0.5, m)
    ef = e.astype(jnp.float32) + jnp.where(big, 1.0, 0.0)
    t = m - 1.0
    s = t / (t + 2.0)
    z2 = s * s
    p = s * (2.0 + z2 * (0.6666666666666666 + z2 * (0.4 + z2 * (
        0.2857142857142857 + z2 * 0.2222222222222222))))
    return ef * 0.6931471805599453 + p


def _make_sc_kernel():
    mesh = plsc.VectorSubcoreMesh(core_axis_name="c", subcore_axis_name="s",
                                  num_cores=2, num_subcores=16)

    @functools.partial(
        pl.kernel,
        out_type=jax.ShapeDtypeStruct((L,), jnp.float32),
        mesh=mesh,
        compiler_params=pltpu.CompilerParams(needs_layout_passes=False),
        scratch_types=[
            pltpu.VMEM((PF,), jnp.float32),
            pltpu.VMEM((PI,), jnp.int32),
            pltpu.VMEM((NL,), jnp.int32),
            pltpu.VMEM((768,), jnp.float32),
            pltpu.VMEM((768,), jnp.float32),
            pltpu.VMEM((704,), jnp.float32),
            pltpu.VMEM((704,), jnp.float32),
            pltpu.VMEM((704,), jnp.float32),
            pltpu.VMEM((64,), jnp.float32),
            pltpu.VMEM((64,), jnp.float32),
            pltpu.VMEM((64,), jnp.float32),
            pltpu.VMEM((L,), jnp.float32),
        ],
    )
    def sc_kernel(pf_hbm, pi_hbm, out_hbm, pf, pi, srcsv, aggR, hT, xlabT,
                  sumLT, hLT, degL, clogv, blogv, outv):
        cid = lax.axis_index("c")
        sid = lax.axis_index("s")

        @pl.when(jnp.logical_and(cid == 0, sid == 0))
        def _body():
            iota = lax.iota(jnp.int32, L)
            zf = jnp.zeros((L,), jnp.float32)
            onef = jnp.ones((L,), jnp.float32)
            lane0 = iota == 0
            lane01 = iota < 2

            def bci(x):
                return jnp.full((L,), x, jnp.int32)

            def bcf(x):
                return jnp.full((L,), x, jnp.float32)

            def lanebc(vec, lane):
                return jnp.take_along_axis(vec, bci(lane), axis=0)

            lt11 = iota < 11
            lt6 = iota < 6
            pat1 = 64 * jnp.where(lt11, iota, iota - 11)
            pat2 = 64 * jnp.where(lt6, iota + 5, 10)
            cpack = 64 * jnp.minimum(iota, 10)

            pltpu.sync_copy(pf_hbm, pf)
            pltpu.sync_copy(pi_hbm, pi)

            for i in range(48):
                aggR[pl.ds(L * i, L)] = zf
            for i in range(44):
                sumLT[pl.ds(L * i, L)] = zf
                xlabT[pl.ds(L * i, L)] = zf
            for i in range(4):
                degL[pl.ds(L * i, L)] = onef

            for i in range(3):
                bp = pi[pl.ds(OFF_BP + L * i, L)]
                srcsv[pl.ds(L * i, L)] = bp % (iota + (L * i + 1))

            def edge_body(e, carry):
                sv = plsc.load_gather(pi, [bci(OFF_ESRC) + bci(e)])
                dv = plsc.load_gather(pi, [bci(OFF_EDST) + bci(e)])
                vals = plsc.load_gather(pf, [sv * L + (OFF_XLR + iota)])
                plsc.addupdate_scatter(aggR, [dv * L + iota], vals)
                return carry

            lax.fori_loop(0, 100, edge_body, jnp.int32(0))

            rdeg = []
            for r in range(3):
                d = plsc.load_gather(aggR, [(iota + L * r) * L + 15]) + 1.0
                rdeg.append(1.0 / d)
            for k in range(15):
                for r in range(3):
                    aggcol = plsc.load_gather(aggR, [(iota + L * r) * L + k])
                    xlcol = pf[pl.ds(OFF_XLT + 48 * k + L * r, L)]
                    hT[pl.ds(48 * k + L * r, L)] = (aggcol + xlcol) * rdeg[r]

            def zbody(j, carry):
                jb = bci(j)
                acc0 = plsc.load_gather(pf, [bci(OFF_BL) + jb])
                acc = [acc0, acc0, acc0]
                for k in range(15):
                    w = plsc.load_gather(pf, [bci(OFF_WL + 64 * k) + jb])
                    for r in range(3):
                        acc[r] = acc[r] + hT[pl.ds(48 * k + L * r, L)] * w
                acc = [jnp.maximum(a, 0.0) for a in acc]
                out = []
                for c in range(NCLS):
                    w = plsc.load_gather(pf, [bci(OFF_WF + c) + jb * L])
                    for r in range(3):
                        out.append(carry[3 * c + r] + acc[r] * w)
                return tuple(out)

            lacc = lax.fori_loop(0, 64, zbody, (zf,) * (3 * NCLS))

            bfv = pf[pl.ds(OFF_BF, L)]
            lT = [[lacc[3 * c + r] + lanebc(bfv, c)
                   for r in range(3)] for c in range(NCLS)]
            lp0 = jnp.float32(0.0)
            for r in range(3):
                mx = lT[0][r]
                for c in range(1, NCLS):
                    mx = jnp.maximum(mx, lT[c][r])
                ex = [jnp.exp(lT[c][r] - mx) for c in range(NCLS)]
                sden = ex[0]
                for c in range(1, NCLS):
                    sden = sden + ex[c]
                inner = zf
                for c in range(NCLS):
                    inner = inner + ex[c] * pf[pl.ds(OFF_OHT + 48 * c + L * r, L)]
                lp0 = lp0 + jnp.sum(_vlog(inner) - _vlog(sden))
                rs = 1.0 / sden
                for c in range(NCLS):
                    p = ex[c] * rs
                    xlabT[pl.ds(64 * c + L * r, L)] = p
                    sumLT[pl.ds(64 * c + L * r, L)] = p
            t10 = xlabT[pl.ds(64 * 10 + STOP, L)]
            xlabT[pl.ds(64 * 10 + STOP, L)] = jnp.where(lane0, onef, t10)
            s10 = sumLT[pl.ds(64 * 10 + STOP, L)]
            sumLT[pl.ds(64 * 10 + STOP, L)] = jnp.where(lane0, onef, s10)

            bvec = pf[pl.ds(OFF_B, L)]
            for q in range(4):
                acc = zf
                for c in range(NCLS):
                    acc = acc + xlabT[pl.ds(64 * c + L * q, L)] * \
                        lanebc(bvec, c)
                blogv[pl.ds(L * q, L)] = acc

            def full_recompute():
                rdq = [1.0 / degL[pl.ds(L * q, L)] for q in range(4)]
                for c in range(NCLS):
                    for q in range(4):
                        off = 64 * c + L * q
                        hLT[pl.ds(off, L)] = sumLT[pl.ds(off, L)] * rdq[q]

                def jbody(j, carry):
                    jb = bci(j)
                    bg = plsc.load_gather(pf, [bci(OFF_BGR) + jb])
                    acc = [bg, bg, bg, bg]
                    for c in range(NCLS):
                        w = plsc.load_gather(pf, [bci(OFF_WGR + 64 * c) + jb])
                        for q in range(4):
                            acc[q] = acc[q] + hLT[pl.ds(64 * c + L * q, L)] * w
                    av = plsc.load_gather(pf, [bci(OFF_A) + jb])
                    return tuple(carry[q] + jnp.maximum(acc[q], 0.0) * av
                                 for q in range(4))

                al = lax.fori_loop(0, 64, jbody, (zf,) * 4)
                for q in range(3):
                    clogv[pl.ds(L * q, L)] = al[q] + blogv[pl.ds(L * q, L)]
                clogv[pl.ds(48, L)] = jnp.where(
                    lane0, al[3] + blogv[pl.ds(48, L)], -1e30)

            lane1 = iota == 1
            for c in range(NCLS):
                x0 = xlabT[pl.ds(64 * c, L)]
                sumLT[pl.ds(64 * c, L)] = sumLT[pl.ds(64 * c, L)] + \
                    jnp.where(lane1, lanebc(x0, 0), 0.0)
            degL[pl.ds(0, L)] = degL[pl.ds(0, L)] + jnp.where(lane1, 1.0, 0.0)
            full_recompute()
            c48 = clogv[pl.ds(STOP, L)]
            clogv[pl.ds(STOP, L)] = jnp.where(lane0, blogv[pl.ds(STOP, L)], c48)
            for c in range(NCLS):
                x0 = xlabT[pl.ds(64 * c, L)]
                sumLT[pl.ds(64 * c, L)] = sumLT[pl.ds(64 * c, L)] - \
                    jnp.where(lane1, lanebc(x0, 0), 0.0)
            degL[pl.ds(0, L)] = degL[pl.ds(0, L)] - jnp.where(lane1, 1.0, 0.0)

            def softmax_nll(lp):
                lv = [clogv[pl.ds(L * q, L)] for q in range(4)]
                mv = jnp.maximum(jnp.maximum(lv[0], lv[1]),
                                 jnp.maximum(lv[2], lv[3]))
                mb = bcf(jnp.max(mv))
                ev = (jnp.exp(lv[0] - mb) + jnp.exp(lv[1] - mb) +
                      jnp.exp(lv[2] - mb) + jnp.exp(lv[3] - mb))
                return lp - mb - _vlog(bcf(jnp.sum(ev))), lv

            def step(idx, last, lp):
                u0 = jnp.max(plsc.load_gather(srcsv, [bci(idx)]))
                stopping = last != u0
                u = jnp.where(stopping, last, u0)
                v = jnp.where(stopping, jnp.int32(STOP), idx + 1)
                lp, _ = softmax_nll(lp)
                vb = bci(v)
                lp = lp + plsc.load_gather(clogv, [vb])
                ub = bci(u)
                guv = jnp.where(lane0, ub, vb)
                suv = jnp.where(lane0, vb, ub)
                for c in range(NCLS):
                    g = plsc.load_gather(xlabT, [bci(64 * c) + guv])
                    plsc.addupdate_scatter(sumLT, [bci(64 * c) + suv], g,
                                           mask=lane01)
                plsc.addupdate_scatter(degL, [guv], onef, mask=lane01)
                rdu = 1.0 / plsc.load_gather(degL, [ub])
                rdv = 1.0 / plsc.load_gather(degL, [vb])
                hu = [plsc.load_gather(sumLT, [bci(64 * c) + ub]) * rdu
                      for c in range(NCLS)]
                hv = [plsc.load_gather(sumLT, [bci(64 * c) + vb]) * rdv
                      for c in range(NCLS)]
                pu = zf
                pv = zf
                for q in range(4):
                    bg = pf[pl.ds(OFF_BGR + L * q, L)]
                    au = bg
                    av = bg
                    for c in range(NCLS):
                        w = pf[pl.ds(OFF_WGR + 64 * c + L * q, L)]
                        au = au + hu[c] * w
                        av = av + hv[c] * w
                    aq = pf[pl.ds(OFF_A + L * q, L)]
                    pu = pu + jnp.maximum(au, 0.0) * aq
                    pv = pv + jnp.maximum(av, 0.0) * aq
                newlog = jnp.where(lane0, bcf(jnp.sum(pu)), bcf(jnp.sum(pv)))
                newlog = newlog + plsc.load_gather(blogv, [guv])
                plsc.store_scatter(clogv, [guv], newlog, mask=lane01)
                idx2 = jnp.where(stopping, idx, idx + 1)
                last2 = jnp.where(stopping, u0, last)
                return idx2, last2, lp

            lp1, lv0 = softmax_nll(bcf(lp0))
            lp1 = lp1 + lanebc(lv0[0], 1)
            swidx = jnp.where(lane0, bci(1), jnp.where(lane1, bci(0), iota))
            for c in range(NCLS):
                x0 = xlabT[pl.ds(64 * c, L)]
                sw = jnp.take_along_axis(x0, swidx, axis=0)
                sumLT[pl.ds(64 * c, L)] = sumLT[pl.ds(64 * c, L)] + \
                    jnp.where(lane01, sw, 0.0)
            degL[pl.ds(0, L)] = degL[pl.ds(0, L)] + jnp.where(lane01, 1.0, 0.0)
            full_recompute()
            idx1 = jnp.int32(1)
            last1 = jnp.int32(0)

            def wcond(carry):
                idx, _, _, it = carry
                return jnp.logical_and(idx < N_EDGES, it < N_STEPS)

            def wbody(carry):
                idx, last, lp, it = carry
                i2, l2, lp2 = step(idx, last, lp)
                return (i2, l2, lp2, it + 1)

            _, _, lpf, _ = lax.while_loop(
                wcond, wbody, (idx1, last1, lp1, jnp.int32(1)))

            outv[...] = lpf
            pltpu.sync_copy(outv, out_hbm)

    return sc_kernel


_SC_KERNEL_CACHE = []


def _get_sc_kernel():
    if not _SC_KERNEL_CACHE:
        _SC_KERNEL_CACHE.append(_make_sc_kernel())
    return _SC_KERNEL_CACHE[0]


def kernel(x_p, edge_index_p, x_l, edge_index_l, bfs_parent, Wp, bp, Wl, bl,
           Wgr, bgr, Wf, bf, Wg, bg):
    f32 = jnp.float32
    xlr = jnp.concatenate([x_l.astype(f32),
                           jnp.ones((NL, 1), f32)], axis=1)
    xlt = xlr.T
    oht = x_l[:, 4:].astype(f32).T
    wl = jnp.concatenate([Wl.astype(f32), jnp.zeros((1, 64), f32)], axis=0)
    wf = jnp.concatenate([Wf.astype(f32), jnp.zeros((64, 5), f32)], axis=1)
    bf16 = jnp.concatenate([bf.astype(f32), jnp.zeros((5,), f32)])
    a_vec = Wg[204:268, 0].astype(f32)
    b_vec = jnp.concatenate([Wg[268:279, 0].astype(f32), jnp.zeros((5,), f32)])
    pf = jnp.concatenate([
        xlr.reshape(-1), xlt.reshape(-1), oht.reshape(-1), wl.reshape(-1),
        bl.astype(f32), wf.reshape(-1), bf16, Wgr.astype(f32).reshape(-1),
        bgr.astype(f32), a_vec, b_vec,
    ])
    i32 = jnp.int32
    esrc = jnp.concatenate([edge_index_l[0].astype(i32),
                            jnp.zeros((12,), i32)])
    edst = jnp.concatenate([edge_index_l[1].astype(i32),
                            jnp.zeros((12,), i32)])
    bpv = jnp.concatenate([bfs_parent.astype(i32), jnp.zeros((1,), i32)])
    pi = jnp.concatenate([esrc, edst, bpv])
    out = _get_sc_kernel()(pf, pi)
    return out[0]
